# baseline (speedup 1.0000x reference)
import functools

import jax
import jax.numpy as jnp
from jax import lax
from jax.experimental import pallas as pl
from jax.experimental.pallas import tpu as pltpu
from jax.experimental.pallas import tpu_sc as plsc

BATCH = 4096
HIST = 200
EMBED_DIM = 64
PAIR_DIM = 128
VOCAB = 1000000
VOCAB_PAIRS = VOCAB // 2
B = BATCH * HIST
NC = 2
NS = 16
NW = NC * NS
BPW = B // NW
BLK = 128
NBLK = BPW // BLK
BB = BATCH // BLK
VBLK = 128
NVFULL = VOCAB // VBLK
VTAIL = VOCAB - NVFULL * VBLK
K1_ITERS = NVFULL // NW + 1

_COMPILER_PARAMS = pltpu.CompilerParams(use_tc_tiling_on_sc=True,
                                        needs_layout_passes=False)


def _build_format_kernel():
  mesh = plsc.VectorSubcoreMesh(core_axis_name="c", subcore_axis_name="s")

  @functools.partial(
      pl.kernel,
      mesh=mesh,
      out_type=jax.ShapeDtypeStruct((VOCAB_PAIRS, PAIR_DIM), jnp.float32),
      scratch_types=[
          pltpu.VMEM((EMBED_DIM, VBLK), jnp.float32),
          pltpu.VMEM((EMBED_DIM, VBLK), jnp.float32),
          pltpu.VMEM((VBLK // 2, PAIR_DIM), jnp.float32),
          pltpu.SemaphoreType.DMA,
          pltpu.SemaphoreType.DMA,
      ],
      compiler_params=_COMPILER_PARAMS,
  )
  def format_kernel(tabt_hbm, tail_hbm, out_hbm, ebuf0, ebuf1, pbuf,
                    sem0, sem1):
    wid = lax.axis_index("s") * NC + lax.axis_index("c")

    def load_start(blk, buf, sem):
      pltpu.async_copy(tabt_hbm.at[:, pl.ds(blk * VBLK, VBLK)], buf, sem)

    def load_wait(blk, buf, sem):
      pltpu.make_async_copy(
          tabt_hbm.at[:, pl.ds(blk * VBLK, VBLK)], buf, sem).wait()

    iota16 = jax.lax.iota(jnp.int32, 16)
    eidx = [iota16 + (16 * k) % 64 for k in range(8)]
    cidx = [iota16 + 16 * k for k in range(8)]

    load_start(wid, ebuf0, sem0)

    def body(i, carry):
      blk = wid + i * NW

      def run(buf, sem, obuf, osem):
        nblk = blk + NW

        @pl.when(nblk < NVFULL)
        def _():
          load_start(nblk, obuf, osem)

        load_wait(blk, buf, sem)

        @plsc.parallel_loop(0, VBLK // 2, unroll=8)
        def trans(p0):
          pmod = jnp.bitwise_and(p0 + iota16, VBLK // 2 - 1)
          vloc0 = pmod + pmod
          vloc1 = vloc0 + 1
          for k in range(8):
            vloc = vloc0 if k < 4 else vloc1
            val = plsc.load_gather(buf, [eidx[k], vloc])
            plsc.store_scatter(pbuf, [pmod, cidx[k]], val)
        pltpu.sync_copy(pbuf, out_hbm.at[pl.ds(blk * (VBLK // 2), VBLK // 2)])

      @pl.when(blk < NVFULL)
      def _():
        @pl.when(i % 2 == 0)
        def _():
          run(ebuf0, sem0, ebuf1, sem1)

        @pl.when(i % 2 == 1)
        def _():
          run(ebuf1, sem1, ebuf0, sem0)

      return carry

    lax.fori_loop(0, K1_ITERS, body, 0)

    @pl.when(wid == 0)
    def _():
      pltpu.sync_copy(tail_hbm, pbuf.at[pl.ds(0, VTAIL // 2)])
      pltpu.sync_copy(pbuf.at[pl.ds(0, VTAIL // 2)],
                      out_hbm.at[pl.ds(NVFULL * (VBLK // 2), VTAIL // 2)])

  return format_kernel


def _build_lookup_kernel():
  mesh = plsc.VectorSubcoreMesh(core_axis_name="c", subcore_axis_name="s")

  @functools.partial(
      pl.kernel,
      mesh=mesh,
      out_type=jax.ShapeDtypeStruct((HIST, EMBED_DIM, BATCH), jnp.float32),
      scratch_types=[
          pltpu.VMEM((BPW,), jnp.int32),
          pltpu.VMEM((BPW,), jnp.int32),
          pltpu.VMEM((BLK, PAIR_DIM), jnp.float32),
          pltpu.VMEM((BLK, PAIR_DIM), jnp.float32),
          pltpu.VMEM((EMBED_DIM, BLK), jnp.float32),
          pltpu.SemaphoreType.DMA,
          pltpu.SemaphoreType.DMA,
      ],
      compiler_params=_COMPILER_PARAMS,
  )
  def lookup_kernel(idx_hbm, table_hbm, out_hbm, idx_v, idx2_v,
                    gbuf0, gbuf1, tbuf, sem0, sem1):
    wid = lax.axis_index("s") * NC + lax.axis_index("c")
    base = wid * BPW
    pltpu.sync_copy(idx_hbm.at[pl.ds(base, BPW)], idx_v)

    def halve(i, carry):
      v = idx_v[pl.ds(i * 16, 16)]
      idx2_v[pl.ds(i * 16, 16)] = lax.shift_right_logical(v, 1)
      return carry

    lax.fori_loop(0, BPW // 16, halve, 0)

    def gather_start(j, buf, sem):
      pltpu.async_copy(
          table_hbm.at[idx2_v.at[pl.ds(j * BLK, BLK)]], buf, sem)

    def gather_wait(j, buf, sem):
      pltpu.make_async_copy(
          table_hbm.at[idx2_v.at[pl.ds(j * BLK, BLK)]], buf, sem).wait()

    gather_start(0, gbuf0, sem0)

    iota16 = jax.lax.iota(jnp.int32, 16)
    iotas = [iota16 + 16 * g for g in range(8)]

    def body(j, carry):
      def run(buf, sem, obuf, osem):
        @pl.when(j + 1 < NBLK)
        def _():
          gather_start(j + 1, obuf, osem)

        gather_wait(j, buf, sem)

        par64 = [
            lax.shift_left(
                jnp.bitwise_and(idx_v[pl.ds(j * BLK + 16 * g, 16)], 1), 6)
            for g in range(8)
        ]

        @plsc.parallel_loop(0, EMBED_DIM, unroll=8)
        def trans(e0):
          emod = jnp.bitwise_and(e0 + iota16, EMBED_DIM - 1)
          for g in range(8):
            val = plsc.load_gather(buf, [iotas[g], par64[g] + emod])
            plsc.store_scatter(tbuf, [emod, iotas[g]], val)

        g_id = wid * NBLK + j
        t = g_id // BB
        b1 = g_id % BB
        pltpu.sync_copy(tbuf, out_hbm.at[t, :, pl.ds(b1 * BLK, BLK)])

      @pl.when(j % 2 == 0)
      def _():
        run(gbuf0, sem0, gbuf1, sem1)

      @pl.when(j % 2 == 1)
      def _():
        run(gbuf1, sem1, gbuf0, sem0)

      return carry

    lax.fori_loop(0, NBLK, body, 0)

  return lookup_kernel


_FORMAT = _build_format_kernel()
_LOOKUP = _build_lookup_kernel()


def kernel(input_x, table):
  idx = input_x.T.reshape(B).astype(jnp.int32)
  tail = table[NVFULL * VBLK:].reshape(VTAIL // 2, PAIR_DIM)
  table_pairs = _FORMAT(table.T, tail)
  out = _LOOKUP(idx, table_pairs)
  return out.transpose(2, 0, 1)


# speedup vs baseline: 1.1846x; 1.1846x over previous
import functools

import jax
import jax.numpy as jnp
from jax import lax
from jax.experimental import pallas as pl
from jax.experimental.pallas import tpu as pltpu
from jax.experimental.pallas import tpu_sc as plsc

BATCH = 4096
HIST = 200
EMBED_DIM = 64
PAIR_DIM = 128
VOCAB = 1000000
VOCAB_PAIRS = VOCAB // 2
B = BATCH * HIST
NC = 2
NS = 16
NW = NC * NS
BPW = B // NW
BLK = 128
NBLK = BPW // BLK
BB = BATCH // BLK
VBLK = 128
NVFULL = VOCAB // VBLK
VTAIL = VOCAB - NVFULL * VBLK
K1_ITERS = NVFULL // NW + 1

_COMPILER_PARAMS = pltpu.CompilerParams(use_tc_tiling_on_sc=True,
                                        needs_layout_passes=False)


def _build_format_kernel():
  mesh = plsc.VectorSubcoreMesh(core_axis_name="c", subcore_axis_name="s")

  @functools.partial(
      pl.kernel,
      mesh=mesh,
      out_type=jax.ShapeDtypeStruct((VOCAB_PAIRS, PAIR_DIM), jnp.float32),
      scratch_types=[
          pltpu.VMEM((EMBED_DIM, VBLK), jnp.float32),
          pltpu.VMEM((EMBED_DIM, VBLK), jnp.float32),
          pltpu.VMEM((VBLK // 2, PAIR_DIM), jnp.float32),
          pltpu.VMEM((VBLK // 2, PAIR_DIM), jnp.float32),
          pltpu.SemaphoreType.DMA,
          pltpu.SemaphoreType.DMA,
          pltpu.SemaphoreType.DMA,
          pltpu.SemaphoreType.DMA,
      ],
      compiler_params=_COMPILER_PARAMS,
  )
  def format_kernel(tabt_hbm, tail_hbm, out_hbm, ebuf0, ebuf1, pbuf0, pbuf1,
                    sem0, sem1, wsem0, wsem1):
    wid = lax.axis_index("s") * NC + lax.axis_index("c")

    def load_start(blk, buf, sem):
      pltpu.async_copy(tabt_hbm.at[:, pl.ds(blk * VBLK, VBLK)], buf, sem)

    def load_wait(blk, buf, sem):
      pltpu.make_async_copy(
          tabt_hbm.at[:, pl.ds(blk * VBLK, VBLK)], buf, sem).wait()

    iota16 = jax.lax.iota(jnp.int32, 16)
    eidx = [iota16 + (16 * k) % 64 for k in range(8)]
    cidx = [iota16 + 16 * k for k in range(8)]

    load_start(wid, ebuf0, sem0)

    def store_start(blk, pbuf, wsem):
      pltpu.async_copy(
          pbuf, out_hbm.at[pl.ds(blk * (VBLK // 2), VBLK // 2)], wsem)

    def store_wait(blk, pbuf, wsem):
      pltpu.make_async_copy(
          pbuf, out_hbm.at[pl.ds(blk * (VBLK // 2), VBLK // 2)], wsem).wait()

    def body(i, carry):
      blk = wid + i * NW

      def run(buf, sem, obuf, osem, pbuf, wsem):
        nblk = blk + NW

        @pl.when(nblk < NVFULL)
        def _():
          load_start(nblk, obuf, osem)

        load_wait(blk, buf, sem)

        @pl.when(i >= 2)
        def _():
          store_wait(blk, pbuf, wsem)

        @plsc.parallel_loop(0, VBLK // 2, unroll=4)
        def trans(p0):
          pmod = jnp.bitwise_and(p0 + iota16, VBLK // 2 - 1)
          vloc0 = pmod + pmod
          vloc1 = vloc0 + 1
          for k in range(8):
            vloc = vloc0 if k < 4 else vloc1
            val = plsc.load_gather(buf, [eidx[k], vloc])
            plsc.store_scatter(pbuf, [pmod, cidx[k]], val)
        store_start(blk, pbuf, wsem)

      @pl.when(blk < NVFULL)
      def _():
        @pl.when(i % 2 == 0)
        def _():
          run(ebuf0, sem0, ebuf1, sem1, pbuf0, wsem0)

        @pl.when(i % 2 == 1)
        def _():
          run(ebuf1, sem1, ebuf0, sem0, pbuf1, wsem1)

      return carry

    lax.fori_loop(0, K1_ITERS, body, 0)

    store_wait(0, pbuf0, wsem0)
    store_wait(0, pbuf1, wsem1)

    @pl.when(wid == 0)
    def _():
      pltpu.sync_copy(tail_hbm, pbuf0.at[pl.ds(0, VTAIL // 2)])
      pltpu.sync_copy(pbuf0.at[pl.ds(0, VTAIL // 2)],
                      out_hbm.at[pl.ds(NVFULL * (VBLK // 2), VTAIL // 2)])

  return format_kernel


def _build_lookup_kernel():
  mesh = plsc.VectorSubcoreMesh(core_axis_name="c", subcore_axis_name="s")

  @functools.partial(
      pl.kernel,
      mesh=mesh,
      out_type=jax.ShapeDtypeStruct((HIST, EMBED_DIM, BATCH), jnp.float32),
      scratch_types=[
          pltpu.VMEM((BPW,), jnp.int32),
          pltpu.VMEM((BPW,), jnp.int32),
          pltpu.VMEM((BLK, PAIR_DIM), jnp.float32),
          pltpu.VMEM((BLK, PAIR_DIM), jnp.float32),
          pltpu.VMEM((EMBED_DIM, BLK), jnp.float32),
          pltpu.VMEM((EMBED_DIM, BLK), jnp.float32),
          pltpu.SemaphoreType.DMA,
          pltpu.SemaphoreType.DMA,
          pltpu.SemaphoreType.DMA,
          pltpu.SemaphoreType.DMA,
      ],
      compiler_params=_COMPILER_PARAMS,
  )
  def lookup_kernel(idx_hbm, table_hbm, out_hbm, idx_v, idx2_v,
                    gbuf0, gbuf1, tbuf0, tbuf1, sem0, sem1, wsem0, wsem1):
    wid = lax.axis_index("s") * NC + lax.axis_index("c")
    base = wid * BPW
    pltpu.sync_copy(idx_hbm.at[pl.ds(base, BPW)], idx_v)

    def halve(i, carry):
      v = idx_v[pl.ds(i * 16, 16)]
      idx2_v[pl.ds(i * 16, 16)] = lax.shift_right_logical(v, 1)
      return carry

    lax.fori_loop(0, BPW // 16, halve, 0)

    def gather_start(j, buf, sem):
      pltpu.async_copy(
          table_hbm.at[idx2_v.at[pl.ds(j * BLK, BLK)]], buf, sem)

    def gather_wait(j, buf, sem):
      pltpu.make_async_copy(
          table_hbm.at[idx2_v.at[pl.ds(j * BLK, BLK)]], buf, sem).wait()

    gather_start(0, gbuf0, sem0)

    iota16 = jax.lax.iota(jnp.int32, 16)
    iotas = [iota16 + 16 * g for g in range(8)]

    def out_slice(j):
      g_id = wid * NBLK + j
      t = g_id // BB
      b1 = g_id % BB
      return out_hbm.at[t, :, pl.ds(b1 * BLK, BLK)]

    def body(j, carry):
      def run(buf, sem, obuf, osem, tbuf, wsem):
        @pl.when(j + 1 < NBLK)
        def _():
          gather_start(j + 1, obuf, osem)

        gather_wait(j, buf, sem)

        @pl.when(j >= 2)
        def _():
          pltpu.make_async_copy(tbuf, out_slice(j), wsem).wait()

        par64 = [
            lax.shift_left(
                jnp.bitwise_and(idx_v[pl.ds(j * BLK + 16 * g, 16)], 1), 6)
            for g in range(8)
        ]

        @plsc.parallel_loop(0, EMBED_DIM, unroll=4)
        def trans(e0):
          emod = jnp.bitwise_and(e0 + iota16, EMBED_DIM - 1)
          for g in range(8):
            val = plsc.load_gather(buf, [iotas[g], par64[g] + emod])
            plsc.store_scatter(tbuf, [emod, iotas[g]], val)

        pltpu.async_copy(tbuf, out_slice(j), wsem)

      @pl.when(j % 2 == 0)
      def _():
        run(gbuf0, sem0, gbuf1, sem1, tbuf0, wsem0)

      @pl.when(j % 2 == 1)
      def _():
        run(gbuf1, sem1, gbuf0, sem0, tbuf1, wsem1)

      return carry

    lax.fori_loop(0, NBLK, body, 0)

    pltpu.make_async_copy(tbuf0, out_slice(NBLK - 2), wsem0).wait()
    pltpu.make_async_copy(tbuf1, out_slice(NBLK - 1), wsem1).wait()

  return lookup_kernel


_FORMAT = _build_format_kernel()
_LOOKUP = _build_lookup_kernel()


def kernel(input_x, table):
  idx = input_x.T.reshape(B).astype(jnp.int32)
  tail = table[NVFULL * VBLK:].reshape(VTAIL // 2, PAIR_DIM)
  table_pairs = _FORMAT(table.T, tail)
  out = _LOOKUP(idx, table_pairs)
  return out.transpose(2, 0, 1)


# speedup vs baseline: 1.2334x; 1.0412x over previous
import functools

import jax
import jax.numpy as jnp
from jax import lax
from jax.experimental import pallas as pl
from jax.experimental.pallas import tpu as pltpu
from jax.experimental.pallas import tpu_sc as plsc

BATCH = 4096
HIST = 200
EMBED_DIM = 64
PAIR_DIM = 128
VOCAB = 1000000
VOCAB_PAIRS = VOCAB // 2
B = BATCH * HIST
NC = 2
NS = 16
NW = NC * NS
BPW = B // NW
BLK = 128
NBLK = BPW // BLK
BB = BATCH // BLK
VBLK = 128
NVFULL = VOCAB // VBLK
VTAIL = VOCAB - NVFULL * VBLK
K1_ITERS = NVFULL // NW + 1

_COMPILER_PARAMS = pltpu.CompilerParams(use_tc_tiling_on_sc=True,
                                        needs_layout_passes=False)


def _build_format_kernel():
  mesh = plsc.VectorSubcoreMesh(core_axis_name="c", subcore_axis_name="s")

  @functools.partial(
      pl.kernel,
      mesh=mesh,
      out_type=jax.ShapeDtypeStruct((VOCAB_PAIRS, PAIR_DIM), jnp.float32),
      scratch_types=[
          pltpu.VMEM((EMBED_DIM, VBLK), jnp.float32),
          pltpu.VMEM((EMBED_DIM, VBLK), jnp.float32),
          pltpu.VMEM((VBLK // 2, PAIR_DIM), jnp.float32),
          pltpu.VMEM((VBLK // 2, PAIR_DIM), jnp.float32),
          pltpu.SemaphoreType.DMA,
          pltpu.SemaphoreType.DMA,
          pltpu.SemaphoreType.DMA,
          pltpu.SemaphoreType.DMA,
      ],
      compiler_params=_COMPILER_PARAMS,
  )
  def format_kernel(tabt_hbm, tail_hbm, out_hbm, ebuf0, ebuf1, pbuf0, pbuf1,
                    sem0, sem1, wsem0, wsem1):
    wid = lax.axis_index("s") * NC + lax.axis_index("c")

    def load_start(blk, buf, sem):
      pltpu.async_copy(tabt_hbm.at[:, pl.ds(blk * VBLK, VBLK)], buf, sem)

    def load_wait(blk, buf, sem):
      pltpu.make_async_copy(
          tabt_hbm.at[:, pl.ds(blk * VBLK, VBLK)], buf, sem).wait()

    iota16 = jax.lax.iota(jnp.int32, 16)
    eidx = [iota16 + (16 * k) % 64 for k in range(8)]
    cidx = [iota16 + 16 * k for k in range(8)]

    load_start(wid, ebuf0, sem0)

    def store_start(blk, pbuf, wsem):
      pltpu.async_copy(
          pbuf, out_hbm.at[pl.ds(blk * (VBLK // 2), VBLK // 2)], wsem)

    def store_wait(blk, pbuf, wsem):
      pltpu.make_async_copy(
          pbuf, out_hbm.at[pl.ds(blk * (VBLK // 2), VBLK // 2)], wsem).wait()

    def body(i, carry):
      blk = wid + i * NW

      def run(buf, sem, obuf, osem, pbuf, wsem):
        nblk = blk + NW

        @pl.when(nblk < NVFULL)
        def _():
          load_start(nblk, obuf, osem)

        load_wait(blk, buf, sem)

        @pl.when(i >= 2)
        def _():
          store_wait(blk, pbuf, wsem)

        @plsc.parallel_loop(0, VBLK // 2, unroll=4)
        def trans(p0):
          pmod = jnp.bitwise_and(p0 + iota16, VBLK // 2 - 1)
          vloc0 = pmod + pmod
          vloc1 = vloc0 + 1
          for k in range(8):
            vloc = vloc0 if k < 4 else vloc1
            val = plsc.load_gather(buf, [eidx[k], vloc])
            plsc.store_scatter(pbuf, [pmod, cidx[k]], val)
        store_start(blk, pbuf, wsem)

      @pl.when(blk < NVFULL)
      def _():
        @pl.when(i % 2 == 0)
        def _():
          run(ebuf0, sem0, ebuf1, sem1, pbuf0, wsem0)

        @pl.when(i % 2 == 1)
        def _():
          run(ebuf1, sem1, ebuf0, sem0, pbuf1, wsem1)

      return carry

    lax.fori_loop(0, K1_ITERS, body, 0)

    store_wait(0, pbuf0, wsem0)
    store_wait(0, pbuf1, wsem1)

    @pl.when(wid == 0)
    def _():
      pltpu.sync_copy(tail_hbm, pbuf0.at[pl.ds(0, VTAIL // 2)])
      pltpu.sync_copy(pbuf0.at[pl.ds(0, VTAIL // 2)],
                      out_hbm.at[pl.ds(NVFULL * (VBLK // 2), VTAIL // 2)])

  return format_kernel


def _build_lookup_kernel():
  mesh = plsc.VectorSubcoreMesh(core_axis_name="c", subcore_axis_name="s")

  @functools.partial(
      pl.kernel,
      mesh=mesh,
      out_type=jax.ShapeDtypeStruct((HIST, EMBED_DIM, BATCH), jnp.float32),
      scratch_types=[
          pltpu.VMEM((BPW,), jnp.int32),
          pltpu.VMEM((BPW,), jnp.int32),
          pltpu.VMEM((BLK, PAIR_DIM), jnp.float32),
          pltpu.VMEM((BLK, PAIR_DIM), jnp.float32),
          pltpu.VMEM((BLK, PAIR_DIM), jnp.float32),
          pltpu.VMEM((EMBED_DIM, BLK), jnp.float32),
          pltpu.VMEM((EMBED_DIM, BLK), jnp.float32),
          pltpu.SemaphoreType.DMA,
          pltpu.SemaphoreType.DMA,
          pltpu.SemaphoreType.DMA,
          pltpu.SemaphoreType.DMA,
          pltpu.SemaphoreType.DMA,
      ],
      compiler_params=_COMPILER_PARAMS,
  )
  def lookup_kernel(idx_hbm, table_hbm, out_hbm, idx_v, idx2_v,
                    gbuf0, gbuf1, gbuf2, tbuf0, tbuf1,
                    sem0, sem1, sem2, wsem0, wsem1):
    wid = lax.axis_index("s") * NC + lax.axis_index("c")
    base = wid * BPW
    pltpu.sync_copy(idx_hbm.at[pl.ds(base, BPW)], idx_v)

    def halve(i, carry):
      v = idx_v[pl.ds(i * 16, 16)]
      idx2_v[pl.ds(i * 16, 16)] = lax.shift_right_logical(v, 1)
      return carry

    lax.fori_loop(0, BPW // 16, halve, 0)

    def gather_start(j, buf, sem):
      pltpu.async_copy(
          table_hbm.at[idx2_v.at[pl.ds(j * BLK, BLK)]], buf, sem)

    def gather_wait(j, buf, sem):
      pltpu.make_async_copy(
          table_hbm.at[idx2_v.at[pl.ds(j * BLK, BLK)]], buf, sem).wait()

    gather_start(0, gbuf0, sem0)
    gather_start(1, gbuf1, sem1)

    iota16 = jax.lax.iota(jnp.int32, 16)
    iotas = [iota16 + 16 * g for g in range(8)]

    def out_slice(j):
      g_id = wid * NBLK + j
      t = g_id // BB
      b1 = g_id % BB
      return out_hbm.at[t, :, pl.ds(b1 * BLK, BLK)]

    def body(j, carry):
      def run(buf, sem, obuf, osem, tbuf, wsem):
        @pl.when(j + 2 < NBLK)
        def _():
          gather_start(j + 2, obuf, osem)

        gather_wait(j, buf, sem)

        @pl.when(j >= 2)
        def _():
          pltpu.make_async_copy(tbuf, out_slice(j), wsem).wait()

        par64 = [
            lax.shift_left(
                jnp.bitwise_and(idx_v[pl.ds(j * BLK + 16 * g, 16)], 1), 6)
            for g in range(8)
        ]

        @plsc.parallel_loop(0, EMBED_DIM, unroll=4)
        def trans(e0):
          emod = jnp.bitwise_and(e0 + iota16, EMBED_DIM - 1)
          for g in range(8):
            val = plsc.load_gather(buf, [iotas[g], par64[g] + emod])
            plsc.store_scatter(tbuf, [emod, iotas[g]], val)

        pltpu.async_copy(tbuf, out_slice(j), wsem)

      gbufs = ((gbuf0, sem0), (gbuf1, sem1), (gbuf2, sem2))
      for r in range(3):
        @pl.when(j % 3 == r)
        def _(r=r):
          buf, sem = gbufs[r]
          obuf, osem = gbufs[(r + 2) % 3]

          @pl.when(j % 2 == 0)
          def _():
            run(buf, sem, obuf, osem, tbuf0, wsem0)

          @pl.when(j % 2 == 1)
          def _():
            run(buf, sem, obuf, osem, tbuf1, wsem1)

      return carry

    lax.fori_loop(0, NBLK, body, 0)

    pltpu.make_async_copy(tbuf0, out_slice(NBLK - 2), wsem0).wait()
    pltpu.make_async_copy(tbuf1, out_slice(NBLK - 1), wsem1).wait()

  return lookup_kernel


_FORMAT = _build_format_kernel()
_LOOKUP = _build_lookup_kernel()


def kernel(input_x, table):
  idx = input_x.T.reshape(B).astype(jnp.int32)
  tail = table[NVFULL * VBLK:].reshape(VTAIL // 2, PAIR_DIM)
  table_pairs = _FORMAT(table.T, tail)
  out = _LOOKUP(idx, table_pairs)
  return out.transpose(2, 0, 1)
